# baseline (speedup 1.0000x reference)
import functools

import jax
import jax.numpy as jnp
from jax import lax
from jax.experimental import pallas as pl
from jax.experimental.pallas import tpu as pltpu
from jax.experimental.pallas import tpu_sc as plsc

T = 0.07
NBG = 4096
K = 4
NPOS = 50
B = 1024
M = 100000
D = 128

MPAD = 100352
MC = 2048
NCHUNK = MPAD // MC
NB = 16
LO0 = -1.002
W1 = 2.004 / NB
W2 = W1 / NB
CAP = 8192
RB = 16
NW = 32
RPW = B // NW
NREG = 3200
NPROP = 84

import numpy as _np
_T1 = [float(_np.float32(LO0 + j * W1)) for j in range(NB)]
_O2 = [float(_np.float32(k * W2)) for k in range(NB + 1)]


def _body_a(pn_ref, bank_ref, dots_ref, cnt_ref):
    i = pl.program_id(0)
    d = lax.dot_general(pn_ref[...], bank_ref[...], (((1,), (1,)), ((), ())),
                        preferred_element_type=jnp.float32)
    col = lax.broadcasted_iota(jnp.int32, (B, MC), 1) + i * MC
    d = jnp.where(col < M, d, -2.0)
    dots_ref[...] = d
    c = jnp.concatenate(
        [jnp.sum((d > t).astype(jnp.float32), axis=1, keepdims=True)
         for t in _T1], axis=1)

    @pl.when(i == 0)
    def _():
        cnt_ref[...] = c

    @pl.when(i > 0)
    def _():
        cnt_ref[...] += c


def _run_a(pn, bank_padded):
    return pl.pallas_call(
        _body_a,
        grid=(NCHUNK,),
        in_specs=[pl.BlockSpec((B, D), lambda i: (0, 0)),
                  pl.BlockSpec((MC, D), lambda i: (i, 0))],
        out_specs=[pl.BlockSpec((B, MC), lambda i: (0, i)),
                   pl.BlockSpec((B, NB), lambda i: (0, 0))],
        out_shape=[jax.ShapeDtypeStruct((B, MPAD), jnp.float32),
                   jax.ShapeDtypeStruct((B, NB), jnp.float32)],
    )(pn, bank_padded)


def _body_b(dots_ref, lo_ref, cnt_ref):
    i = pl.program_id(0)
    d = dots_ref[...]
    lo = lo_ref[...]
    c = jnp.concatenate(
        [jnp.sum((d > (lo + _O2[k + 1])).astype(jnp.float32), axis=1,
                 keepdims=True) for k in range(NB)], axis=1)

    @pl.when(i == 0)
    def _():
        cnt_ref[...] = c

    @pl.when(i > 0)
    def _():
        cnt_ref[...] += c


def _run_b(dots, lo):
    return pl.pallas_call(
        _body_b,
        grid=(NCHUNK,),
        in_specs=[pl.BlockSpec((B, MC), lambda i: (0, i)),
                  pl.BlockSpec((B, 1), lambda i: (0, 0))],
        out_specs=pl.BlockSpec((B, NB), lambda i: (0, 0)),
        out_shape=jax.ShapeDtypeStruct((B, NB), jnp.float32),
    )(dots, lo)


def _sc_compact_body(dots, tau, idx_out, val_out,
                     tau_v, in0, in1, idxb, valb, sem0, sem1):
    wid = lax.axis_index("s") * 2 + lax.axis_index("c")
    r0 = wid * RPW
    pltpu.sync_copy(tau.at[pl.ds(r0, RPW)], tau_v.at[pl.ds(0, RPW)])

    def row_body(rr, carry):
        r = r0 + rr
        tau_r = tau_v[pl.ds(rr, 16)][0]

        bufs = (in0, in1)
        sems = (sem0, sem1)
        pltpu.make_async_copy(dots.at[r, pl.ds(0, MC)], in0, sem0).start()
        off = jnp.int32(0)
        for c in range(NCHUNK):
            sl = c & 1
            if c + 1 < NCHUNK:
                pltpu.make_async_copy(dots.at[r, pl.ds((c + 1) * MC, MC)],
                                      bufs[(c + 1) & 1], sems[(c + 1) & 1]).start()
            pltpu.make_async_copy(dots.at[r, pl.ds(c * MC, MC)],
                                  bufs[sl], sems[sl]).wait()
            buf = bufs[sl]
            cbase = c * MC

            def vec_body(k16, off):
                v = buf[pl.ds(k16 * 16, 16)]
                mask = v > tau_r
                iv = lax.iota(jnp.int32, 16) + (cbase + k16 * 16)
                keys = mask.astype(jnp.int32)
                civ = plsc.sort_key_val(keys, iv, descending=True)[1]
                cv = plsc.sort_key_val(keys, v, descending=True)[1]
                o = jnp.minimum(off, CAP - 16)
                idxb[pl.ds(o, 16)] = civ
                valb[pl.ds(o, 16)] = cv
                return off + plsc.all_reduce_population_count(mask)[0]

            off = lax.fori_loop(0, MC // 16, vec_body, off)
        pltpu.sync_copy(idxb, idx_out.at[r])
        pltpu.sync_copy(valb, val_out.at[r])
        return carry

    lax.fori_loop(0, RPW, row_body, 0)


def _run_compact(dots, tau):
    mesh = plsc.VectorSubcoreMesh(core_axis_name="c", subcore_axis_name="s")
    f = pl.kernel(
        _sc_compact_body,
        out_type=[jax.ShapeDtypeStruct((B, CAP), jnp.int32),
                  jax.ShapeDtypeStruct((B, CAP), jnp.float32)],
        mesh=mesh,
        compiler_params=pltpu.CompilerParams(needs_layout_passes=False),
        scratch_types=[pltpu.VMEM((RPW + 16,), jnp.float32),
                       pltpu.VMEM((MC,), jnp.float32),
                       pltpu.VMEM((MC,), jnp.float32),
                       pltpu.VMEM((CAP,), jnp.int32),
                       pltpu.VMEM((CAP,), jnp.float32),
                       pltpu.SemaphoreType.DMA,
                       pltpu.SemaphoreType.DMA],
    )
    return f(dots, tau)


NPP = 96
MW = 48


def _sc_gather_body(neigh16f, pi, dots, neighs_out, pos_out, self_out,
                    piv, matf, kidsnf, rowbuf, nodes, posc, selfb, semg, semr):
    wid = lax.axis_index("s") * 2 + lax.axis_index("c")
    r0 = wid * RPW
    pltpu.sync_copy(pi.at[pl.ds(r0, RPW)], piv.at[pl.ds(0, RPW)])

    def do_round(src_base, dst_base):
        def fire(rr, carry):
            if src_base is None:
                node = piv[pl.ds(rr, 16)][0]
            else:
                node = matf[pl.ds(src_base + rr, 16)][0]
            pltpu.make_async_copy(neigh16f.at[pl.ds(node * 16, 16)],
                                  kidsnf.at[pl.ds(rr * 16, 16)], semg).start()
            return carry
        lax.fori_loop(0, RPW, fire, 0)
        pltpu.make_async_copy(neigh16f.at[pl.ds(0, RPW * 16)],
                              kidsnf, semg).wait()
        for j in range(K):
            for g in range(RPW // 16):
                src = (lax.iota(jnp.int32, 16) + g * 16) * 16 + j
                matf[pl.ds((dst_base + j) * MW + g * 16, 16)] = (
                    plsc.load_gather(kidsnf, [src]))

    do_round(None, 0)
    for t in range(20):
        do_round(t * MW, 4 + 4 * t)

    def row_body(rr, carry):
        r = r0 + rr
        pltpu.make_async_copy(dots.at[r], rowbuf, semr).start()
        pltpu.make_async_copy(dots.at[r], rowbuf, semr).wait()
        for g in range(NPP // 16):
            colidx = (lax.iota(jnp.int32, 16) + g * 16) * MW + rr
            nv = plsc.load_gather(matf, [colidx])
            nodes[pl.ds(g * 16, 16)] = nv
            pv = plsc.load_gather(rowbuf, [jnp.clip(nv, 0, MPAD - 1)])
            posc[pl.ds(g * 16, 16)] = pv
        pltpu.sync_copy(nodes, neighs_out.at[r])
        pltpu.sync_copy(posc, pos_out.at[r])
        spi = piv[pl.ds(rr, 16)][0]
        selfv = plsc.load_gather(rowbuf, [jnp.full((16,), 0, jnp.int32) + spi])
        plsc.store_scatter(selfb, [jnp.full((16,), 0, jnp.int32) + rr], selfv,
                           mask=lax.iota(jnp.int32, 16) == 0)
        return carry
    lax.fori_loop(0, RPW, row_body, 0)
    pltpu.sync_copy(selfb.at[pl.ds(0, RPW)], self_out.at[pl.ds(r0, RPW)])


def _run_gather(neigh16f, pi, dots):
    mesh = plsc.VectorSubcoreMesh(core_axis_name="c", subcore_axis_name="s")
    f = pl.kernel(
        _sc_gather_body,
        out_type=[jax.ShapeDtypeStruct((B, NPP), jnp.int32),
                  jax.ShapeDtypeStruct((B, NPP), jnp.float32),
                  jax.ShapeDtypeStruct((B,), jnp.float32)],
        mesh=mesh,
        compiler_params=pltpu.CompilerParams(needs_layout_passes=False),
        scratch_types=[pltpu.VMEM((RPW + 16,), jnp.int32),
                       pltpu.VMEM((NPP * MW,), jnp.int32),
                       pltpu.VMEM((RPW * 16,), jnp.int32),
                       pltpu.VMEM((MPAD,), jnp.float32),
                       pltpu.VMEM((NPP,), jnp.int32),
                       pltpu.VMEM((NPP,), jnp.float32),
                       pltpu.VMEM((MW,), jnp.float32),
                       pltpu.SemaphoreType.DMA,
                       pltpu.SemaphoreType.DMA],
    )
    return f(neigh16f, pi, dots)


def _ce(e, ix, j, k, lane):
    n = e.shape[1]
    up = (lane & j) == 0
    desc = (lane & k) == 0
    pe = jnp.where(up, pltpu.roll(e, n - j, 1), pltpu.roll(e, j, 1))
    pi_ = jnp.where(up, pltpu.roll(ix, n - j, 1), pltpu.roll(ix, j, 1))
    beats = (e > pe) | ((e == pe) & (ix < pi_))
    keep = beats == (up == desc)
    return jnp.where(keep, e, pe), jnp.where(keep, ix, pi_)


def _bitonic_desc(e, ix):
    n = e.shape[1]
    lane = lax.broadcasted_iota(jnp.int32, e.shape, 1)
    k = 2
    while k <= n:
        j = k // 2
        while j >= 1:
            e, ix = _ce(e, ix, j, k, lane)
            j //= 2
        k *= 2
    return e, ix


def _bitonic_asc_vals(v):
    n = v.shape[1]
    lane = lax.broadcasted_iota(jnp.int32, v.shape, 1)
    k = 2
    while k <= n:
        j = k // 2
        desc = (lane & k) != 0
        while j >= 1:
            up = (lane & j) == 0
            pv = jnp.where(up, pltpu.roll(v, n - j, 1), pltpu.roll(v, j, 1))
            beats = (v > pv) | ((v == pv) & up)
            keep = beats == (up == desc)
            v = jnp.where(keep, v, pv)
            j //= 2
        k *= 2
    return v


def _body_e(val_ref, idx_ref, n_ref, self_ref, pos_ref, pi_ref,
            bg_ref, nei_ref, la_ref, lb_ref):
    n = n_ref[...]
    lane = lax.broadcasted_iota(jnp.int32, (RB, CAP), 1)
    valid = lane < n
    e = jnp.where(valid, jnp.exp(val_ref[...] / T), -1.0)
    ix = jnp.where(valid, idx_ref[...], lane + MPAD)
    e, ix = _bitonic_desc(e, ix)
    bg_ref[...] = ix[:, :NBG]
    bg_sum = jnp.sum(e[:, :NBG], axis=1, keepdims=True)
    self_e = jnp.exp(self_ref[...] / T)
    la_ref[...] = jnp.log(self_e / bg_sum + 1e-07)

    p = _bitonic_asc_vals(pos_ref[...])
    hp = jnp.sum(jnp.exp(p[:, :NPOS] / T), axis=1, keepdims=True)
    lb_ref[...] = jnp.log(hp / (bg_sum - self_e) + 1e-07)

    top8 = ix[:, :8]
    cond = top8 == pi_ref[...]
    backup = jnp.broadcast_to(top8[:, 4:5], top8.shape)
    nei_ex = jnp.where(cond, backup, top8)
    nei_ref[...] = jnp.concatenate([nei_ex[:, :4], ix[:, 4:128]], axis=1)


def _run_e(cand_val, cand_idx, n_above, self_d, pos_d, pi):
    grid = (B // RB,)
    return pl.pallas_call(
        _body_e,
        grid=grid,
        in_specs=[pl.BlockSpec((RB, CAP), lambda i: (i, 0)),
                  pl.BlockSpec((RB, CAP), lambda i: (i, 0)),
                  pl.BlockSpec((RB, 1), lambda i: (i, 0)),
                  pl.BlockSpec((RB, 1), lambda i: (i, 0)),
                  pl.BlockSpec((RB, 128), lambda i: (i, 0)),
                  pl.BlockSpec((RB, 1), lambda i: (i, 0))],
        out_specs=[pl.BlockSpec((RB, NBG), lambda i: (i, 0)),
                   pl.BlockSpec((RB, 128), lambda i: (i, 0)),
                   pl.BlockSpec((RB, 1), lambda i: (i, 0)),
                   pl.BlockSpec((RB, 1), lambda i: (i, 0))],
        out_shape=[jax.ShapeDtypeStruct((B, NBG), jnp.int32),
                   jax.ShapeDtypeStruct((B, 128), jnp.int32),
                   jax.ShapeDtypeStruct((B, 1), jnp.float32),
                   jax.ShapeDtypeStruct((B, 1), jnp.float32)],
    )(cand_val, cand_idx, n_above, self_d, pos_d, pi)


def _body_g0(pi_ref, pit_ref, nei_ref, out_ref):
    eq = pi_ref[...] == pit_ref[...]
    jmat = lax.broadcasted_iota(jnp.int32, (B, B), 1)
    lastj = jnp.max(jnp.where(eq, jmat, -1), axis=1, keepdims=True)
    onehot = (jmat == lastj).astype(jnp.float32)
    nei_f = nei_ref[...].astype(jnp.float32)
    out = lax.dot_general(onehot, nei_f, (((1,), (0,)), ((), ())),
                          preferred_element_type=jnp.float32,
                          precision=jax.lax.Precision.HIGHEST)
    out_ref[...] = out.astype(jnp.int32)


def _run_g0(pi, nei128):
    return pl.pallas_call(
        _body_g0,
        in_specs=[pl.BlockSpec((B, 1), lambda: (0, 0)),
                  pl.BlockSpec((1, B), lambda: (0, 0)),
                  pl.BlockSpec((B, 128), lambda: (0, 0))],
        out_specs=pl.BlockSpec((B, 128), lambda: (0, 0)),
        out_shape=jax.ShapeDtypeStruct((B, 128), jnp.int32),
    )(pi[:, None], pi[None, :], nei128)


def _sc_scatter_body(neigh_flat, pi, nei4f, upd, piv, neivf, regf):
    wid = lax.axis_index("s") * 2 + lax.axis_index("c")
    base = jnp.minimum(wid * NREG, M - NREG)
    pltpu.sync_copy(pi, piv)
    pltpu.sync_copy(nei4f, neivf)
    pltpu.sync_copy(neigh_flat.at[pl.ds(base * K, NREG * K)], regf)

    def apply_body(g, carry):
        i16 = lax.iota(jnp.int32, 16) + g * 16
        p = piv[pl.ds(g * 16, 16)]
        q = p - base
        inm = (q >= 0) & (q < NREG)
        qc = jnp.clip(q, 0, NREG - 1)
        for col in range(K):
            vals = plsc.load_gather(neivf, [i16 * K + col])
            plsc.store_scatter(regf, [qc * K + col], vals, mask=inm)
        return carry

    lax.fori_loop(0, B // 16, apply_body, 0)
    pltpu.sync_copy(regf, upd.at[pl.ds(base * K, NREG * K)])


def _run_scatter(neigh_flat, pi, nei4f):
    mesh = plsc.VectorSubcoreMesh(core_axis_name="c", subcore_axis_name="s")
    f = pl.kernel(
        _sc_scatter_body,
        out_type=jax.ShapeDtypeStruct((M * K,), jnp.int32),
        mesh=mesh,
        compiler_params=pltpu.CompilerParams(needs_layout_passes=False),
        scratch_types=[pltpu.VMEM((B,), jnp.int32),
                       pltpu.VMEM((B * K,), jnp.int32),
                       pltpu.VMEM((NREG * K,), jnp.int32)],
    )
    return f(neigh_flat, pi, nei4f)


def kernel(points, point_indices, bank, neigh):
    bank_padded = jnp.pad(bank, ((0, MPAD - M), (0, 0)))
    norm_points = points / jnp.sqrt(jnp.sum(points ** 2, axis=1, keepdims=True))

    dots, cnt = _run_a(norm_points, bank_padded)

    jstar = jnp.sum((cnt >= NBG).astype(jnp.int32), axis=1) - 1
    los = jnp.array(_T1, jnp.float32)
    lo = los[jstar][:, None]

    cnt2 = _run_b(dots, lo)
    base_cnt = jnp.take_along_axis(cnt, jstar[:, None], axis=1)
    ladder = jnp.concatenate([base_cnt, cnt2], axis=1)
    kstar = jnp.sum((ladder >= NBG).astype(jnp.int32), axis=1) - 1
    offs = jnp.array(_O2, jnp.float32)
    tau = (lo[:, 0] + offs[kstar]).astype(jnp.float32)
    n_above = jnp.take_along_axis(ladder, kstar[:, None], axis=1).astype(jnp.int32)

    cand_idx, cand_val = _run_compact(dots, tau)

    neigh_flat = jnp.reshape(neigh, (-1,))
    neigh16f = jnp.reshape(jnp.pad(neigh, ((0, 0), (0, 16 - K))), (-1,))
    neighs_p, pos_p, self_d = _run_gather(neigh16f, point_indices, dots)
    neighs = neighs_p[:, :NPROP]
    pos_pad = jnp.concatenate(
        [pos_p[:, :NPROP],
         jnp.full((B, 128 - NPROP), jnp.inf, jnp.float32)], axis=1)

    bg_idx, nei128, la, lb = _run_e(
        cand_val, cand_idx, n_above, self_d[:, None], pos_pad,
        point_indices[:, None])

    lossA = -jnp.mean(la)
    lossB = -jnp.mean(lb)

    nei_last = _run_g0(point_indices, nei128)
    nei4f = jnp.reshape(nei_last[:, :K], (-1,))
    updated_neigh = jnp.reshape(
        _run_scatter(neigh_flat, point_indices, nei4f), (M, K))

    return lossA, lossB, bg_idx, neighs, updated_neigh


# speedup vs baseline: 1.0555x; 1.0555x over previous
import functools

import jax
import jax.numpy as jnp
from jax import lax
from jax.experimental import pallas as pl
from jax.experimental.pallas import tpu as pltpu
from jax.experimental.pallas import tpu_sc as plsc

T = 0.07
NBG = 4096
K = 4
NPOS = 50
B = 1024
M = 100000
D = 128

MPAD = 100352
MC = 2048
NCHUNK = MPAD // MC
NB = 16
LO0 = -1.002
W1 = 2.004 / NB
W2 = W1 / NB
CAP = 8192
RB = 8
NW = 32
RPW = B // NW
NREG = 3200
NPROP = 84

import numpy as _np
_T1 = [float(_np.float32(LO0 + j * W1)) for j in range(NB)]
_O2 = [float(_np.float32(k * W2)) for k in range(NB + 1)]


def _body_a(pn_ref, bank_ref, dots_ref, cnt_ref):
    i = pl.program_id(0)
    d = lax.dot_general(pn_ref[...], bank_ref[...], (((1,), (1,)), ((), ())),
                        preferred_element_type=jnp.float32)
    col = lax.broadcasted_iota(jnp.int32, (B, MC), 1) + i * MC
    d = jnp.where(col < M, d, -2.0)
    dots_ref[...] = d
    c = jnp.concatenate(
        [jnp.sum((d > t).astype(jnp.float32), axis=1, keepdims=True)
         for t in _T1], axis=1)

    @pl.when(i == 0)
    def _():
        cnt_ref[...] = c

    @pl.when(i > 0)
    def _():
        cnt_ref[...] += c


def _run_a(pn, bank_padded):
    return pl.pallas_call(
        _body_a,
        grid=(NCHUNK,),
        in_specs=[pl.BlockSpec((B, D), lambda i: (0, 0)),
                  pl.BlockSpec((MC, D), lambda i: (i, 0))],
        out_specs=[pl.BlockSpec((B, MC), lambda i: (0, i)),
                   pl.BlockSpec((B, NB), lambda i: (0, 0))],
        out_shape=[jax.ShapeDtypeStruct((B, MPAD), jnp.float32),
                   jax.ShapeDtypeStruct((B, NB), jnp.float32)],
    )(pn, bank_padded)


def _body_b(dots_ref, lo_ref, cnt_ref):
    i = pl.program_id(0)
    d = dots_ref[...]
    lo = lo_ref[...]
    c = jnp.concatenate(
        [jnp.sum((d > (lo + _O2[k + 1])).astype(jnp.float32), axis=1,
                 keepdims=True) for k in range(NB)], axis=1)

    @pl.when(i == 0)
    def _():
        cnt_ref[...] = c

    @pl.when(i > 0)
    def _():
        cnt_ref[...] += c


def _run_b(dots, lo):
    return pl.pallas_call(
        _body_b,
        grid=(NCHUNK,),
        in_specs=[pl.BlockSpec((B, MC), lambda i: (0, i)),
                  pl.BlockSpec((B, 1), lambda i: (0, 0))],
        out_specs=pl.BlockSpec((B, NB), lambda i: (0, 0)),
        out_shape=jax.ShapeDtypeStruct((B, NB), jnp.float32),
    )(dots, lo)


def _sc_compact_body(dots, tau, idx_out, val_out,
                     tau_v, in0, in1, idxb, valb, sem0, sem1):
    wid = lax.axis_index("s") * 2 + lax.axis_index("c")
    r0 = wid * RPW
    pltpu.sync_copy(tau.at[pl.ds(r0, RPW)], tau_v.at[pl.ds(0, RPW)])

    def row_body(rr, carry):
        r = r0 + rr
        tau_r = tau_v[pl.ds(rr, 16)][0]

        bufs = (in0, in1)
        sems = (sem0, sem1)
        pltpu.make_async_copy(dots.at[r, pl.ds(0, MC)], in0, sem0).start()
        off = jnp.int32(0)
        for c in range(NCHUNK):
            sl = c & 1
            if c + 1 < NCHUNK:
                pltpu.make_async_copy(dots.at[r, pl.ds((c + 1) * MC, MC)],
                                      bufs[(c + 1) & 1], sems[(c + 1) & 1]).start()
            pltpu.make_async_copy(dots.at[r, pl.ds(c * MC, MC)],
                                  bufs[sl], sems[sl]).wait()
            buf = bufs[sl]
            cbase = c * MC

            def vec_body(k16, off):
                v = buf[pl.ds(k16 * 16, 16)]
                mask = v > tau_r
                iv = lax.iota(jnp.int32, 16) + (cbase + k16 * 16)
                keys = mask.astype(jnp.int32)
                civ = plsc.sort_key_val(keys, iv, descending=True)[1]
                cv = plsc.sort_key_val(keys, v, descending=True)[1]
                o = jnp.minimum(off, CAP - 16)
                idxb[pl.ds(o, 16)] = civ
                valb[pl.ds(o, 16)] = cv
                return off + plsc.all_reduce_population_count(mask)[0]

            off = lax.fori_loop(0, MC // 16, vec_body, off)
        pltpu.sync_copy(idxb, idx_out.at[r])
        pltpu.sync_copy(valb, val_out.at[r])
        return carry

    lax.fori_loop(0, RPW, row_body, 0)


def _run_compact(dots, tau):
    mesh = plsc.VectorSubcoreMesh(core_axis_name="c", subcore_axis_name="s")
    f = pl.kernel(
        _sc_compact_body,
        out_type=[jax.ShapeDtypeStruct((B, CAP), jnp.int32),
                  jax.ShapeDtypeStruct((B, CAP), jnp.float32)],
        mesh=mesh,
        compiler_params=pltpu.CompilerParams(needs_layout_passes=False),
        scratch_types=[pltpu.VMEM((RPW + 16,), jnp.float32),
                       pltpu.VMEM((MC,), jnp.float32),
                       pltpu.VMEM((MC,), jnp.float32),
                       pltpu.VMEM((CAP,), jnp.int32),
                       pltpu.VMEM((CAP,), jnp.float32),
                       pltpu.SemaphoreType.DMA,
                       pltpu.SemaphoreType.DMA],
    )
    return f(dots, tau)


NPP = 96
MW = 48


def _sc_gather_body(neigh16f, pi, dots, neighs_out, pos_out, self_out,
                    piv, matf, kidsnf, rowbuf, nodes, posc, selfb, semg, semr):
    wid = lax.axis_index("s") * 2 + lax.axis_index("c")
    r0 = wid * RPW
    pltpu.sync_copy(pi.at[pl.ds(r0, RPW)], piv.at[pl.ds(0, RPW)])

    def do_round(src_base, dst_base):
        def fire(rr, carry):
            if src_base is None:
                node = piv[pl.ds(rr, 16)][0]
            else:
                node = matf[pl.ds(src_base + rr, 16)][0]
            pltpu.make_async_copy(neigh16f.at[pl.ds(node * 16, 16)],
                                  kidsnf.at[pl.ds(rr * 16, 16)], semg).start()
            return carry
        lax.fori_loop(0, RPW, fire, 0)
        pltpu.make_async_copy(neigh16f.at[pl.ds(0, RPW * 16)],
                              kidsnf, semg).wait()
        for j in range(K):
            for g in range(RPW // 16):
                src = (lax.iota(jnp.int32, 16) + g * 16) * 16 + j
                matf[pl.ds((dst_base + j) * MW + g * 16, 16)] = (
                    plsc.load_gather(kidsnf, [src]))

    do_round(None, 0)
    for t in range(20):
        do_round(t * MW, 4 + 4 * t)

    def row_body(rr, carry):
        r = r0 + rr
        pltpu.make_async_copy(dots.at[r], rowbuf, semr).start()
        pltpu.make_async_copy(dots.at[r], rowbuf, semr).wait()
        for g in range(NPP // 16):
            colidx = (lax.iota(jnp.int32, 16) + g * 16) * MW + rr
            nv = plsc.load_gather(matf, [colidx])
            nodes[pl.ds(g * 16, 16)] = nv
            pv = plsc.load_gather(rowbuf, [jnp.clip(nv, 0, MPAD - 1)])
            posc[pl.ds(g * 16, 16)] = pv
        pltpu.sync_copy(nodes, neighs_out.at[r])
        pltpu.sync_copy(posc, pos_out.at[r])
        spi = piv[pl.ds(rr, 16)][0]
        selfv = plsc.load_gather(rowbuf, [jnp.full((16,), 0, jnp.int32) + spi])
        plsc.store_scatter(selfb, [jnp.full((16,), 0, jnp.int32) + rr], selfv,
                           mask=lax.iota(jnp.int32, 16) == 0)
        return carry
    lax.fori_loop(0, RPW, row_body, 0)
    pltpu.sync_copy(selfb.at[pl.ds(0, RPW)], self_out.at[pl.ds(r0, RPW)])


def _run_gather(neigh16f, pi, dots):
    mesh = plsc.VectorSubcoreMesh(core_axis_name="c", subcore_axis_name="s")
    f = pl.kernel(
        _sc_gather_body,
        out_type=[jax.ShapeDtypeStruct((B, NPP), jnp.int32),
                  jax.ShapeDtypeStruct((B, NPP), jnp.float32),
                  jax.ShapeDtypeStruct((B,), jnp.float32)],
        mesh=mesh,
        compiler_params=pltpu.CompilerParams(needs_layout_passes=False),
        scratch_types=[pltpu.VMEM((RPW + 16,), jnp.int32),
                       pltpu.VMEM((NPP * MW,), jnp.int32),
                       pltpu.VMEM((RPW * 16,), jnp.int32),
                       pltpu.VMEM((MPAD,), jnp.float32),
                       pltpu.VMEM((NPP,), jnp.int32),
                       pltpu.VMEM((NPP,), jnp.float32),
                       pltpu.VMEM((MW,), jnp.float32),
                       pltpu.SemaphoreType.DMA,
                       pltpu.SemaphoreType.DMA],
    )
    return f(neigh16f, pi, dots)


def _ce(e, ix, j, k, lane):
    n = e.shape[1]
    up = (lane & j) == 0
    desc = (lane & k) == 0
    pe = jnp.where(up, pltpu.roll(e, n - j, 1), pltpu.roll(e, j, 1))
    pi_ = jnp.where(up, pltpu.roll(ix, n - j, 1), pltpu.roll(ix, j, 1))
    beats = (e > pe) | ((e == pe) & (ix < pi_))
    keep = beats == (up == desc)
    return jnp.where(keep, e, pe), jnp.where(keep, ix, pi_)


def _bitonic_desc(e, ix):
    n = e.shape[1]
    lane = lax.broadcasted_iota(jnp.int32, e.shape, 1)
    k = 2
    while k <= n:
        j = k // 2
        while j >= 1:
            e, ix = _ce(e, ix, j, k, lane)
            j //= 2
        k *= 2
    return e, ix


def _bitonic_asc_vals(v):
    n = v.shape[1]
    lane = lax.broadcasted_iota(jnp.int32, v.shape, 1)
    k = 2
    while k <= n:
        j = k // 2
        desc = (lane & k) != 0
        while j >= 1:
            up = (lane & j) == 0
            pv = jnp.where(up, pltpu.roll(v, n - j, 1), pltpu.roll(v, j, 1))
            beats = (v > pv) | ((v == pv) & up)
            keep = beats == (up == desc)
            v = jnp.where(keep, v, pv)
            j //= 2
        k *= 2
    return v


def _body_e(val_ref, idx_ref, n_ref, self_ref, pos_ref, pi_ref,
            bg_ref, nei_ref, la_ref, lb_ref):
    n = n_ref[...]
    lane = lax.broadcasted_iota(jnp.int32, (RB, CAP), 1)
    valid = lane < n
    e = jnp.where(valid, jnp.exp(val_ref[...] / T), -1.0)
    ix = jnp.where(valid, idx_ref[...], lane + MPAD)
    e, ix = _bitonic_desc(e, ix)
    bg_ref[...] = ix[:, :NBG]
    bg_sum = jnp.sum(e[:, :NBG], axis=1, keepdims=True)
    self_e = jnp.exp(self_ref[...] / T)
    la_ref[...] = jnp.log(self_e / bg_sum + 1e-07)

    p = _bitonic_asc_vals(pos_ref[...])
    hp = jnp.sum(jnp.exp(p[:, :NPOS] / T), axis=1, keepdims=True)
    lb_ref[...] = jnp.log(hp / (bg_sum - self_e) + 1e-07)

    top8 = ix[:, :8]
    cond = top8 == pi_ref[...]
    backup = jnp.broadcast_to(top8[:, 4:5], top8.shape)
    nei_ex = jnp.where(cond, backup, top8)
    nei_ref[...] = jnp.concatenate([nei_ex[:, :4], ix[:, 4:128]], axis=1)


def _run_e(cand_val, cand_idx, n_above, self_d, pos_d, pi):
    grid = (B // RB,)
    return pl.pallas_call(
        _body_e,
        grid=grid,
        in_specs=[pl.BlockSpec((RB, CAP), lambda i: (i, 0)),
                  pl.BlockSpec((RB, CAP), lambda i: (i, 0)),
                  pl.BlockSpec((RB, 1), lambda i: (i, 0)),
                  pl.BlockSpec((RB, 1), lambda i: (i, 0)),
                  pl.BlockSpec((RB, 128), lambda i: (i, 0)),
                  pl.BlockSpec((RB, 1), lambda i: (i, 0))],
        out_specs=[pl.BlockSpec((RB, NBG), lambda i: (i, 0)),
                   pl.BlockSpec((RB, 128), lambda i: (i, 0)),
                   pl.BlockSpec((RB, 1), lambda i: (i, 0)),
                   pl.BlockSpec((RB, 1), lambda i: (i, 0))],
        out_shape=[jax.ShapeDtypeStruct((B, NBG), jnp.int32),
                   jax.ShapeDtypeStruct((B, 128), jnp.int32),
                   jax.ShapeDtypeStruct((B, 1), jnp.float32),
                   jax.ShapeDtypeStruct((B, 1), jnp.float32)],
    )(cand_val, cand_idx, n_above, self_d, pos_d, pi)


def _body_g0(pi_ref, pit_ref, nei_ref, out_ref):
    eq = pi_ref[...] == pit_ref[...]
    jmat = lax.broadcasted_iota(jnp.int32, (B, B), 1)
    lastj = jnp.max(jnp.where(eq, jmat, -1), axis=1, keepdims=True)
    onehot = (jmat == lastj).astype(jnp.float32)
    nei_f = nei_ref[...].astype(jnp.float32)
    out = lax.dot_general(onehot, nei_f, (((1,), (0,)), ((), ())),
                          preferred_element_type=jnp.float32,
                          precision=jax.lax.Precision.HIGHEST)
    out_ref[...] = out.astype(jnp.int32)


def _run_g0(pi, nei128):
    return pl.pallas_call(
        _body_g0,
        in_specs=[pl.BlockSpec((B, 1), lambda: (0, 0)),
                  pl.BlockSpec((1, B), lambda: (0, 0)),
                  pl.BlockSpec((B, 128), lambda: (0, 0))],
        out_specs=pl.BlockSpec((B, 128), lambda: (0, 0)),
        out_shape=jax.ShapeDtypeStruct((B, 128), jnp.int32),
    )(pi[:, None], pi[None, :], nei128)


def _sc_scatter_body(neigh_flat, pi, nei4f, upd, piv, neivf, regf):
    wid = lax.axis_index("s") * 2 + lax.axis_index("c")
    base = jnp.minimum(wid * NREG, M - NREG)
    pltpu.sync_copy(pi, piv)
    pltpu.sync_copy(nei4f, neivf)
    pltpu.sync_copy(neigh_flat.at[pl.ds(base * K, NREG * K)], regf)

    def apply_body(g, carry):
        i16 = lax.iota(jnp.int32, 16) + g * 16
        p = piv[pl.ds(g * 16, 16)]
        q = p - base
        inm = (q >= 0) & (q < NREG)
        qc = jnp.clip(q, 0, NREG - 1)
        for col in range(K):
            vals = plsc.load_gather(neivf, [i16 * K + col])
            plsc.store_scatter(regf, [qc * K + col], vals, mask=inm)
        return carry

    lax.fori_loop(0, B // 16, apply_body, 0)
    pltpu.sync_copy(regf, upd.at[pl.ds(base * K, NREG * K)])


def _run_scatter(neigh_flat, pi, nei4f):
    mesh = plsc.VectorSubcoreMesh(core_axis_name="c", subcore_axis_name="s")
    f = pl.kernel(
        _sc_scatter_body,
        out_type=jax.ShapeDtypeStruct((M * K,), jnp.int32),
        mesh=mesh,
        compiler_params=pltpu.CompilerParams(needs_layout_passes=False),
        scratch_types=[pltpu.VMEM((B,), jnp.int32),
                       pltpu.VMEM((B * K,), jnp.int32),
                       pltpu.VMEM((NREG * K,), jnp.int32)],
    )
    return f(neigh_flat, pi, nei4f)


def kernel(points, point_indices, bank, neigh):
    bank_padded = jnp.pad(bank, ((0, MPAD - M), (0, 0)))
    norm_points = points / jnp.sqrt(jnp.sum(points ** 2, axis=1, keepdims=True))

    dots, cnt = _run_a(norm_points, bank_padded)

    jstar = jnp.sum((cnt >= NBG).astype(jnp.int32), axis=1) - 1
    los = jnp.array(_T1, jnp.float32)
    lo = los[jstar][:, None]

    cnt2 = _run_b(dots, lo)
    base_cnt = jnp.take_along_axis(cnt, jstar[:, None], axis=1)
    ladder = jnp.concatenate([base_cnt, cnt2], axis=1)
    kstar = jnp.sum((ladder >= NBG).astype(jnp.int32), axis=1) - 1
    offs = jnp.array(_O2, jnp.float32)
    tau = (lo[:, 0] + offs[kstar]).astype(jnp.float32)
    n_above = jnp.take_along_axis(ladder, kstar[:, None], axis=1).astype(jnp.int32)

    cand_idx, cand_val = _run_compact(dots, tau)

    neigh_flat = jnp.reshape(neigh, (-1,))
    neigh16f = jnp.reshape(jnp.pad(neigh, ((0, 0), (0, 16 - K))), (-1,))
    neighs_p, pos_p, self_d = _run_gather(neigh16f, point_indices, dots)
    neighs = neighs_p[:, :NPROP]
    pos_pad = jnp.concatenate(
        [pos_p[:, :NPROP],
         jnp.full((B, 128 - NPROP), jnp.inf, jnp.float32)], axis=1)

    bg_idx, nei128, la, lb = _run_e(
        cand_val, cand_idx, n_above, self_d[:, None], pos_pad,
        point_indices[:, None])

    lossA = -jnp.mean(la)
    lossB = -jnp.mean(lb)

    nei_last = _run_g0(point_indices, nei128)
    nei4f = jnp.reshape(nei_last[:, :K], (-1,))
    updated_neigh = jnp.reshape(
        _run_scatter(neigh_flat, point_indices, nei4f), (M, K))

    return lossA, lossB, bg_idx, neighs, updated_neigh
